# baseline (speedup 1.0000x reference)
import functools

import jax
import jax.numpy as jnp
from jax import lax
from jax.experimental import pallas as pl
from jax.experimental.pallas import tpu as pltpu
from jax.experimental.pallas import tpu_sc as plsc

N_VERTS = 5023
N_CROP = 1787
BATCH = 4096
NUM_CORES = 2
NUM_SUBCORES = 16
NW = NUM_CORES * NUM_SUBCORES
JB = 128
BB = 128
N_JBLK = (N_CROP + JB - 1) // JB
IDX_PAD = N_JBLK * JB
LANES = 16
N_TASKS = 3 * N_JBLK


def _make_gather():
    mesh = plsc.VectorSubcoreMesh(core_axis_name="c", subcore_axis_name="s")

    @functools.partial(
        pl.kernel,
        mesh=mesh,
        compiler_params=pltpu.CompilerParams(
            needs_layout_passes=False, disable_bounds_checks=True),
        out_type=jax.ShapeDtypeStruct((3, BATCH, N_CROP), jnp.float32),
        scratch_types=[
            pltpu.VMEM((IDX_PAD,), jnp.int32),
            pltpu.VMEM((2, JB, BB), jnp.float32),
            pltpu.VMEM((2, BB, JB), jnp.float32),
            pltpu.SemaphoreType.DMA((2,)),
            pltpu.SemaphoreType.DMA((2,)),
        ],
    )
    def gather_kernel(xt, vidx_hbm, out_hbm, idx_v, ibufs, obufs,
                      sem_in, sem_out):
        wid = lax.axis_index("s") * NUM_CORES + lax.axis_index("c")
        b0 = wid * BB
        pltpu.sync_copy(vidx_hbm, idx_v)
        iota = lax.iota(jnp.int32, LANES)
        i32 = jnp.int32

        def start_in(par, c, jb):
            pltpu.make_async_copy(
                xt.at[c, :, pl.ds(b0, BB)].at[
                    idx_v.at[pl.ds(jb * JB, JB)]],
                ibufs.at[par], sem_in.at[par]).start()

        def wait_in(par):
            pltpu.make_async_copy(
                xt.at[0, :, pl.ds(b0, BB)].at[idx_v.at[pl.ds(0, JB)]],
                ibufs.at[par], sem_in.at[par]).wait()

        def start_out(par, c, jb):
            joff = pl.multiple_of(jb * JB, JB)
            pltpu.make_async_copy(
                obufs.at[par],
                out_hbm.at[c, pl.ds(b0, BB), pl.ds(joff, JB)],
                sem_out.at[par]).start()

        def wait_out(par):
            pltpu.make_async_copy(
                obufs.at[par],
                out_hbm.at[0, pl.ds(b0, BB), pl.ds(0, JB)],
                sem_out.at[par]).wait()

        def transpose(par):
            pvec = jnp.full((LANES,), par, i32)

            def body(b, _):
                bvec = jnp.full((LANES,), b, i32)
                for m in range(JB // LANES):
                    vals = plsc.load_gather(
                        ibufs, [pvec, iota + (m * LANES), bvec])
                    obufs[par, b, pl.ds(m * LANES, LANES)] = vals
                return 0

            lax.fori_loop(0, BB, body, 0)

        start_in(0, 0, 0)
        start_in(1, 0, 1)

        def step(t, carry):
            c, jb, c2, jb2 = carry
            par = lax.rem(t, 2)
            wait_in(par)

            @pl.when(t >= 2)
            def _():
                wait_out(par)

            transpose(par)
            start_out(par, c, jb)

            @pl.when(t + 2 < N_TASKS)
            def _():
                start_in(par, c2, jb2)

            jb_n = jb + 1
            wrap = jb_n == N_JBLK
            c_n = lax.select(wrap, c + 1, c)
            jb_n = lax.select(wrap, 0, jb_n)
            jb2_n = jb2 + 1
            wrap2 = jb2_n == N_JBLK
            c2_n = lax.select(wrap2, c2 + 1, c2)
            jb2_n = lax.select(wrap2, 0, jb2_n)
            return c_n, jb_n, c2_n, jb2_n

        lax.fori_loop(0, N_TASKS, step,
                      (i32(0), i32(0), i32(0), i32(2)))
        wait_out(0)
        wait_out(1)

    return gather_kernel


_gather = _make_gather()


def kernel(x, vidx):
    vidx32 = vidx.astype(jnp.int32)
    vpad = jnp.concatenate(
        [vidx32, jnp.zeros((IDX_PAD - N_CROP,), jnp.int32)])
    xt = x.transpose(2, 1, 0)
    out = _gather(xt, vpad)
    return out.transpose(1, 2, 0)


# speedup vs baseline: 1.2099x; 1.2099x over previous
import functools

import jax
import jax.numpy as jnp
from jax import lax
from jax.experimental import pallas as pl
from jax.experimental.pallas import tpu as pltpu
from jax.experimental.pallas import tpu_sc as plsc

N_VERTS = 5023
N_CROP = 1787
BATCH = 4096
NUM_CORES = 2
NUM_SUBCORES = 16
NW = NUM_CORES * NUM_SUBCORES
JB = 128
BB = 128
N_JBLK = (N_CROP + JB - 1) // JB
IDX_PAD = N_JBLK * JB
LANES = 16
N_TASKS = 3 * N_JBLK


def _make_gather():
    mesh = plsc.VectorSubcoreMesh(core_axis_name="c", subcore_axis_name="s")

    @functools.partial(
        pl.kernel,
        mesh=mesh,
        compiler_params=pltpu.CompilerParams(
            needs_layout_passes=False, disable_bounds_checks=True),
        out_type=jax.ShapeDtypeStruct((3, BATCH, N_CROP), jnp.float32),
        scratch_types=[
            pltpu.VMEM((IDX_PAD,), jnp.int32),
            pltpu.VMEM((2, JB, BB), jnp.float32),
            pltpu.VMEM((2, BB, JB), jnp.float32),
            pltpu.SemaphoreType.DMA((2,)),
            pltpu.SemaphoreType.DMA((2,)),
        ],
    )
    def gather_kernel(xt, vidx_hbm, out_hbm, idx_v, ibufs, obufs,
                      sem_in, sem_out):
        wid = lax.axis_index("s") * NUM_CORES + lax.axis_index("c")
        b0 = wid * BB
        pltpu.sync_copy(vidx_hbm, idx_v)
        iota = lax.iota(jnp.int32, LANES)
        i32 = jnp.int32

        def start_in(par, c, jb):
            pltpu.make_async_copy(
                xt.at[c, :, pl.ds(b0, BB)].at[
                    idx_v.at[pl.ds(jb * JB, JB)]],
                ibufs.at[par], sem_in.at[par]).start()

        def wait_in(par):
            pltpu.make_async_copy(
                xt.at[0, :, pl.ds(b0, BB)].at[idx_v.at[pl.ds(0, JB)]],
                ibufs.at[par], sem_in.at[par]).wait()

        def start_out(par, c, jb):
            joff = pl.multiple_of(jb * JB, JB)
            pltpu.make_async_copy(
                obufs.at[par],
                out_hbm.at[c, pl.ds(b0, BB), pl.ds(joff, JB)],
                sem_out.at[par]).start()

        def wait_out(par):
            pltpu.make_async_copy(
                obufs.at[par],
                out_hbm.at[0, pl.ds(b0, BB), pl.ds(0, JB)],
                sem_out.at[par]).wait()

        bvecs = [iota + (bc * LANES) for bc in range(BB // LANES)]

        def transpose(par):
            ibuf = ibufs.at[par]
            obuf = obufs.at[par]

            def body(j, _):
                jv = jnp.full((LANES,), j, i32)
                for bc in range(BB // LANES):
                    vals = ibuf[j, pl.ds(bc * LANES, LANES)]
                    plsc.store_scatter(obuf, [bvecs[bc], jv], vals)
                return 0

            lax.fori_loop(0, JB, body, 0, unroll=2)

        def bump(c, jb):
            jb_n = jb + 1
            wrap = jb_n == N_JBLK
            return lax.select(wrap, c + 1, c), lax.select(wrap, 0, jb_n)

        start_in(0, 0, 0)
        start_in(1, 0, 1)

        def step(t2, carry):
            c, jb, c2, jb2 = carry
            for par in range(2):
                t = t2 * 2 + par
                wait_in(par)

                @pl.when(t >= 2)
                def _():
                    wait_out(par)

                transpose(par)
                start_out(par, c, jb)

                @pl.when(t + 2 < N_TASKS)
                def _():
                    start_in(par, c2, jb2)

                c, jb = bump(c, jb)
                c2, jb2 = bump(c2, jb2)
            return c, jb, c2, jb2

        lax.fori_loop(0, N_TASKS // 2, step,
                      (i32(0), i32(0), i32(0), i32(2)))
        wait_out(0)
        wait_out(1)

    return gather_kernel


_gather = _make_gather()


def kernel(x, vidx):
    vidx32 = vidx.astype(jnp.int32)
    vpad = jnp.concatenate(
        [vidx32, jnp.zeros((IDX_PAD - N_CROP,), jnp.int32)])
    xt = x.transpose(2, 1, 0)
    out = _gather(xt, vpad)
    return out.transpose(1, 2, 0)


# speedup vs baseline: 1.6469x; 1.3612x over previous
import functools

import jax
import jax.numpy as jnp
from jax import lax
from jax.experimental import pallas as pl
from jax.experimental.pallas import tpu as pltpu
from jax.experimental.pallas import tpu_sc as plsc

N_VERTS = 5023
N_CROP = 1787
BATCH = 4096
NUM_CORES = 2
NUM_SUBCORES = 16
NW = NUM_CORES * NUM_SUBCORES
JB = 128
BB = 128
N_JBLK = (N_CROP + JB - 1) // JB
IDX_PAD = N_JBLK * JB
LANES = 16
N_TASKS = 3 * N_JBLK


def _make_gather():
    mesh = plsc.VectorSubcoreMesh(core_axis_name="c", subcore_axis_name="s")

    @functools.partial(
        pl.kernel,
        mesh=mesh,
        compiler_params=pltpu.CompilerParams(
            needs_layout_passes=False, disable_bounds_checks=True),
        out_type=jax.ShapeDtypeStruct((3, BATCH, N_CROP), jnp.float32),
        scratch_types=[
            pltpu.VMEM((IDX_PAD,), jnp.int32),
            pltpu.VMEM((2, JB, BB), jnp.float32),
            pltpu.VMEM((2, BB, JB), jnp.float32),
            pltpu.SemaphoreType.DMA((2,)),
            pltpu.SemaphoreType.DMA((2,)),
        ],
    )
    def gather_kernel(xt, vidx_hbm, out_hbm, idx_v, ibufs, obufs,
                      sem_in, sem_out):
        wid = lax.axis_index("s") * NUM_CORES + lax.axis_index("c")
        b0 = wid * BB
        pltpu.sync_copy(vidx_hbm, idx_v)
        iota = lax.iota(jnp.int32, LANES)
        i32 = jnp.int32

        def start_in(par, c, jb):
            pltpu.make_async_copy(
                xt.at[c, :, pl.ds(b0, BB)].at[
                    idx_v.at[pl.ds(jb * JB, JB)]],
                ibufs.at[par], sem_in.at[par]).start()

        def wait_in(par):
            pltpu.make_async_copy(
                xt.at[0, :, pl.ds(b0, BB)].at[idx_v.at[pl.ds(0, JB)]],
                ibufs.at[par], sem_in.at[par]).wait()

        def start_out(par, c, jb):
            joff = pl.multiple_of(jb * JB, JB)
            pltpu.make_async_copy(
                obufs.at[par],
                out_hbm.at[c, pl.ds(b0, BB), pl.ds(joff, JB)],
                sem_out.at[par]).start()

        def wait_out(par):
            pltpu.make_async_copy(
                obufs.at[par],
                out_hbm.at[0, pl.ds(b0, BB), pl.ds(0, JB)],
                sem_out.at[par]).wait()

        bvecs = [iota + (bc * LANES) for bc in range(BB // LANES)]

        def transpose(par):
            ibuf = ibufs.at[par]
            obuf = obufs.at[par]

            @plsc.parallel_loop(0, JB, unroll=2)
            def _(j):
                jv = jnp.full((LANES,), j, i32)
                for bc in range(BB // LANES):
                    vals = ibuf[j, pl.ds(bc * LANES, LANES)]
                    plsc.store_scatter(obuf, [bvecs[bc], jv], vals)

        def bump(c, jb):
            jb_n = jb + 1
            wrap = jb_n == N_JBLK
            return lax.select(wrap, c + 1, c), lax.select(wrap, 0, jb_n)

        start_in(0, 0, 0)
        start_in(1, 0, 1)

        def step(t2, carry):
            c, jb, c2, jb2 = carry
            for par in range(2):
                t = t2 * 2 + par
                wait_in(par)

                @pl.when(t >= 2)
                def _():
                    wait_out(par)

                transpose(par)
                start_out(par, c, jb)

                @pl.when(t + 2 < N_TASKS)
                def _():
                    start_in(par, c2, jb2)

                c, jb = bump(c, jb)
                c2, jb2 = bump(c2, jb2)
            return c, jb, c2, jb2

        lax.fori_loop(0, N_TASKS // 2, step,
                      (i32(0), i32(0), i32(0), i32(2)))
        wait_out(0)
        wait_out(1)

    return gather_kernel


_gather = _make_gather()


def kernel(x, vidx):
    vidx32 = vidx.astype(jnp.int32)
    vpad = jnp.concatenate(
        [vidx32, jnp.zeros((IDX_PAD - N_CROP,), jnp.int32)])
    xt = x.transpose(2, 1, 0)
    out = _gather(xt, vpad)
    return out.transpose(1, 2, 0)
